# baseline (device time: 49274 ns/iter reference)
import jax
import jax.numpy as jnp
from jax import lax
from jax.experimental import pallas as pl
from jax.experimental.pallas import tpu as pltpu

N_DEV = 4
B_PER = 2
SQ = 128
D = 512
H_PER = 8
DH = 64
ROWS = B_PER * SQ


def kernel(x, Wq, Wo, Wk, Wv):
    def body(x_ref, wq_ref, wo_ref, wk_ref, wv_ref, out_ref,
             xall, contribs, recvbuf, attn_ref,
             wq16, wk16, wv16, wo16,
             ag_send, ag_recv, rs_send, rs_recv):
        my = lax.axis_index("i")
        left = lax.rem(my + N_DEV - 1, N_DEV)
        right = lax.rem(my + 1, N_DEV)

        barrier = pltpu.get_barrier_semaphore()
        for nbr in (left, right):
            pl.semaphore_signal(
                barrier, inc=1, device_id=(nbr,),
                device_id_type=pl.DeviceIdType.MESH,
            )
        pl.semaphore_wait(barrier, 2)

        xall[0] = x_ref[...]

        def ag_hop(h):
            return pltpu.make_async_remote_copy(
                src_ref=xall.at[h],
                dst_ref=xall.at[h + 1],
                send_sem=ag_send.at[h],
                recv_sem=ag_recv.at[h],
                device_id=(right,),
                device_id_type=pl.DeviceIdType.MESH,
            )

        def rs_step(s):
            return pltpu.make_async_remote_copy(
                src_ref=contribs.at[s + 1],
                dst_ref=recvbuf.at[s],
                send_sem=rs_send.at[s],
                recv_sem=rs_recv.at[s],
                device_id=(right,),
                device_id_type=pl.DeviceIdType.MESH,
            )

        wq16[...] = wq_ref[...].astype(jnp.bfloat16)
        wk16[...] = wk_ref[...].astype(jnp.bfloat16)
        wv16[...] = wv_ref[...].astype(jnp.bfloat16)
        wo16[...] = wo_ref[...].astype(jnp.bfloat16)

        def contribution(r):
            xc = xall[r].astype(jnp.bfloat16)
            q = jnp.dot(xc, wq16[...], preferred_element_type=jnp.float32)
            k = jnp.dot(xc, wk16[...], preferred_element_type=jnp.float32)
            v = jnp.dot(xc, wv16[...], preferred_element_type=jnp.float32)
            k16 = k.astype(jnp.bfloat16)
            v16 = v.astype(jnp.bfloat16)
            for b in range(B_PER):
                rsl = slice(b * SQ, (b + 1) * SQ)
                for hh in range(H_PER):
                    csl = slice(hh * DH, (hh + 1) * DH)
                    qh = (q[rsl, csl] * 0.125).astype(jnp.bfloat16)
                    kh = k16[rsl, csl]
                    vh = v16[rsl, csl]
                    s = lax.dot_general(
                        qh, kh, (((1,), (1,)), ((), ())),
                        preferred_element_type=jnp.float32,
                    )
                    m = jnp.max(s, axis=-1, keepdims=True)
                    p = jnp.exp(s - m)
                    lsum = jnp.sum(p, axis=-1, keepdims=True)
                    o = jnp.dot(
                        p.astype(jnp.bfloat16), vh,
                        preferred_element_type=jnp.float32,
                    ) / lsum
                    attn_ref[rsl, csl] = o.astype(jnp.bfloat16)
            return jnp.dot(
                attn_ref[...], wo16[...], preferred_element_type=jnp.float32
            )

        ag0 = ag_hop(0)
        ag0.start()
        contribs[0] = contribution(0)

        ag0.wait_recv()
        ag1 = ag_hop(1)
        ag1.start()
        contribs[1] = contribution(1)

        rs0 = rs_step(0)
        rs0.start()
        ag1.wait_recv()
        ag2 = ag_hop(2)
        ag2.start()
        contribs[2] = contribution(2)

        rs0.wait_recv()
        contribs[2] = contribs[2] + recvbuf[0]
        rs1 = rs_step(1)
        rs1.start()
        ag2.wait_recv()
        contribs[3] = contribution(3)

        rs1.wait_recv()
        contribs[3] = contribs[3] + recvbuf[1]
        rs2 = rs_step(2)
        rs2.start()
        rs2.wait_recv()
        out_ref[...] = contribs[0] + recvbuf[N_DEV - 2]

        for d in (ag0, ag1, ag2, rs0, rs1, rs2):
            d.wait_send()

    x2 = x.reshape(ROWS, D)
    out = pl.pallas_call(
        body,
        out_shape=jax.ShapeDtypeStruct((ROWS, D), jnp.float32),
        in_specs=[pl.BlockSpec(memory_space=pltpu.VMEM)] * 5,
        out_specs=pl.BlockSpec(memory_space=pltpu.VMEM),
        scratch_shapes=[
            pltpu.VMEM((N_DEV, ROWS, D), jnp.float32),
            pltpu.VMEM((N_DEV, ROWS, D), jnp.float32),
            pltpu.VMEM((N_DEV - 1, ROWS, D), jnp.float32),
            pltpu.VMEM((ROWS, D), jnp.bfloat16),
            pltpu.VMEM((D, D), jnp.bfloat16),
            pltpu.VMEM((D, D), jnp.bfloat16),
            pltpu.VMEM((D, D), jnp.bfloat16),
            pltpu.VMEM((D, D), jnp.bfloat16),
            pltpu.SemaphoreType.DMA((N_DEV - 1,)),
            pltpu.SemaphoreType.DMA((N_DEV - 1,)),
            pltpu.SemaphoreType.DMA((N_DEV - 1,)),
            pltpu.SemaphoreType.DMA((N_DEV - 1,)),
        ],
        compiler_params=pltpu.CompilerParams(collective_id=0),
    )(x2, Wq, Wo, Wk, Wv)
    return out.reshape(B_PER, SQ, D)


# device time: 33655 ns/iter; 1.4641x vs baseline; 1.4641x over previous
import jax
import jax.numpy as jnp
from jax import lax
from jax.experimental import pallas as pl
from jax.experimental.pallas import tpu as pltpu

N_DEV = 4
B_PER = 2
SQ = 128
D = 512
H_PER = 8
DH = 64
ROWS = B_PER * SQ


def kernel(x, Wq, Wo, Wk, Wv):
    def body(x_ref, wq_ref, wo_ref, wk_ref, wv_ref, out_ref,
             xall, contribs, recvbuf, sendbuf, attn_ref,
             wqkv16, wo16,
             ag_send, ag_recv, rs_send, rs_recv):
        my = lax.axis_index("i")
        left = lax.rem(my + N_DEV - 1, N_DEV)
        right = lax.rem(my + 1, N_DEV)

        barrier = pltpu.get_barrier_semaphore()
        for nbr in (left, right):
            pl.semaphore_signal(
                barrier, inc=1, device_id=(nbr,),
                device_id_type=pl.DeviceIdType.MESH,
            )
        pl.semaphore_wait(barrier, 2)

        xall[0] = x_ref[...].astype(jnp.bfloat16)

        def ag_hop(h):
            return pltpu.make_async_remote_copy(
                src_ref=xall.at[h],
                dst_ref=xall.at[h + 1],
                send_sem=ag_send.at[h],
                recv_sem=ag_recv.at[h],
                device_id=(right,),
                device_id_type=pl.DeviceIdType.MESH,
            )

        def rs_step(s):
            return pltpu.make_async_remote_copy(
                src_ref=sendbuf.at[s],
                dst_ref=recvbuf.at[s],
                send_sem=rs_send.at[s],
                recv_sem=rs_recv.at[s],
                device_id=(right,),
                device_id_type=pl.DeviceIdType.MESH,
            )

        wqkv16[:, 0 * D:1 * D] = wq_ref[...].astype(jnp.bfloat16)
        wqkv16[:, 1 * D:2 * D] = wk_ref[...].astype(jnp.bfloat16)
        wqkv16[:, 2 * D:3 * D] = wv_ref[...].astype(jnp.bfloat16)
        wo16[...] = wo_ref[...].astype(jnp.bfloat16)

        def contribution(r):
            qkv = jnp.dot(
                xall[r], wqkv16[...], preferred_element_type=jnp.float32
            )
            qkv16 = qkv.astype(jnp.bfloat16)
            for b in range(B_PER):
                rsl = slice(b * SQ, (b + 1) * SQ)
                for hh in range(H_PER):
                    qsl = slice(hh * DH, (hh + 1) * DH)
                    ksl = slice(D + hh * DH, D + (hh + 1) * DH)
                    vsl = slice(2 * D + hh * DH, 2 * D + (hh + 1) * DH)
                    qh = qkv16[rsl, qsl]
                    kh = qkv16[rsl, ksl]
                    vh = qkv16[rsl, vsl]
                    s = lax.dot_general(
                        qh, kh, (((1,), (1,)), ((), ())),
                        preferred_element_type=jnp.float32,
                    )
                    p = jnp.exp(s * 0.125)
                    lsum = jnp.sum(p, axis=-1, keepdims=True)
                    o = jnp.dot(
                        p.astype(jnp.bfloat16), vh,
                        preferred_element_type=jnp.float32,
                    ) / lsum
                    attn_ref[rsl, hh * DH:(hh + 1) * DH] = o.astype(jnp.bfloat16)
            return jnp.dot(
                attn_ref[...], wo16[...], preferred_element_type=jnp.float32
            )

        ag0 = ag_hop(0)
        ag0.start()
        contribs[0] = contribution(0)

        ag0.wait_recv()
        ag1 = ag_hop(1)
        ag1.start()
        contribs[1] = contribution(1)

        sendbuf[0] = contribs[1].astype(jnp.bfloat16)
        rs0 = rs_step(0)
        rs0.start()
        ag1.wait_recv()
        ag2 = ag_hop(2)
        ag2.start()
        contribs[2] = contribution(2)

        rs0.wait_recv()
        sendbuf[1] = (contribs[2] + recvbuf[0].astype(jnp.float32)).astype(
            jnp.bfloat16)
        rs1 = rs_step(1)
        rs1.start()
        ag2.wait_recv()
        contribs[3] = contribution(3)

        rs1.wait_recv()
        sendbuf[2] = (contribs[3] + recvbuf[1].astype(jnp.float32)).astype(
            jnp.bfloat16)
        rs2 = rs_step(2)
        rs2.start()
        rs2.wait_recv()
        out_ref[...] = contribs[0] + recvbuf[N_DEV - 2].astype(jnp.float32)

        for d in (ag0, ag1, ag2, rs0, rs1, rs2):
            d.wait_send()

    x2 = x.reshape(ROWS, D)
    out = pl.pallas_call(
        body,
        out_shape=jax.ShapeDtypeStruct((ROWS, D), jnp.float32),
        in_specs=[pl.BlockSpec(memory_space=pltpu.VMEM)] * 5,
        out_specs=pl.BlockSpec(memory_space=pltpu.VMEM),
        scratch_shapes=[
            pltpu.VMEM((N_DEV, ROWS, D), jnp.bfloat16),
            pltpu.VMEM((N_DEV, ROWS, D), jnp.float32),
            pltpu.VMEM((N_DEV - 1, ROWS, D), jnp.bfloat16),
            pltpu.VMEM((N_DEV - 1, ROWS, D), jnp.bfloat16),
            pltpu.VMEM((ROWS, D), jnp.bfloat16),
            pltpu.VMEM((D, 3 * D), jnp.bfloat16),
            pltpu.VMEM((D, D), jnp.bfloat16),
            pltpu.SemaphoreType.DMA((N_DEV - 1,)),
            pltpu.SemaphoreType.DMA((N_DEV - 1,)),
            pltpu.SemaphoreType.DMA((N_DEV - 1,)),
            pltpu.SemaphoreType.DMA((N_DEV - 1,)),
        ],
        compiler_params=pltpu.CompilerParams(collective_id=0),
    )(x2, Wq, Wo, Wk, Wv)
    return out.reshape(B_PER, SQ, D)


# device time: 32397 ns/iter; 1.5209x vs baseline; 1.0388x over previous
import jax
import jax.numpy as jnp
from jax import lax
from jax.experimental import pallas as pl
from jax.experimental.pallas import tpu as pltpu

N_DEV = 4
B_PER = 2
SQ = 128
D = 512
H_PER = 8
DH = 64
ROWS = B_PER * SQ


def kernel(x, Wq, Wo, Wk, Wv):
    def body(x_ref, wq_ref, wo_ref, wk_ref, wv_ref, out_ref,
             xall, contribs, recvbuf, sendbuf,
             wqkv16, wo16,
             ag_send, ag_recv, rs_send, rs_recv):
        my = lax.axis_index("i")
        left = lax.rem(my + N_DEV - 1, N_DEV)
        right = lax.rem(my + 1, N_DEV)

        barrier = pltpu.get_barrier_semaphore()
        for nbr in (left, right):
            pl.semaphore_signal(
                barrier, inc=1, device_id=(nbr,),
                device_id_type=pl.DeviceIdType.MESH,
            )
        pl.semaphore_wait(barrier, 2)

        xall[0] = x_ref[...].astype(jnp.bfloat16)

        def ag_hop(h):
            return pltpu.make_async_remote_copy(
                src_ref=xall.at[h],
                dst_ref=xall.at[h + 1],
                send_sem=ag_send.at[h],
                recv_sem=ag_recv.at[h],
                device_id=(right,),
                device_id_type=pl.DeviceIdType.MESH,
            )

        def rs_step(s):
            return pltpu.make_async_remote_copy(
                src_ref=sendbuf.at[s],
                dst_ref=recvbuf.at[s],
                send_sem=rs_send.at[s],
                recv_sem=rs_recv.at[s],
                device_id=(right,),
                device_id_type=pl.DeviceIdType.MESH,
            )

        wqkv16[:, 0 * D:1 * D] = wq_ref[...].astype(jnp.bfloat16)
        wqkv16[:, 1 * D:2 * D] = wk_ref[...].astype(jnp.bfloat16)
        wqkv16[:, 2 * D:3 * D] = wv_ref[...].astype(jnp.bfloat16)
        wo16[...] = wo_ref[...].astype(jnp.bfloat16)

        def contribution(r):
            qkv = jnp.dot(
                xall[r], wqkv16[...], preferred_element_type=jnp.float32
            )
            qkv16 = qkv.astype(jnp.bfloat16)

            def heads(i):
                parts = []
                for hh in range(H_PER):
                    csl = slice(i * D + hh * DH, i * D + (hh + 1) * DH)
                    parts.append(qkv16[:, csl].reshape(1, B_PER, SQ, DH))
                return jnp.concatenate(parts, axis=0).reshape(
                    H_PER * B_PER, SQ, DH)

            q4 = heads(0)
            k4 = heads(1)
            v4 = heads(2)
            s = lax.dot_general(
                q4, k4, (((2,), (2,)), ((0,), (0,))),
                preferred_element_type=jnp.float32,
            )
            p = jnp.exp(s * 0.125)
            lsum = jnp.sum(p, axis=-1, keepdims=True)
            o = lax.dot_general(
                p.astype(jnp.bfloat16), v4, (((2,), (1,)), ((0,), (0,))),
                preferred_element_type=jnp.float32,
            ) / lsum
            attn = o.reshape(H_PER, B_PER, SQ, DH).transpose(
                1, 2, 0, 3).reshape(ROWS, D)
            return jnp.dot(
                attn.astype(jnp.bfloat16), wo16[...],
                preferred_element_type=jnp.float32,
            )

        ag0 = ag_hop(0)
        ag0.start()
        contribs[0] = contribution(0)

        ag0.wait_recv()
        ag1 = ag_hop(1)
        ag1.start()
        contribs[1] = contribution(1)

        sendbuf[0] = contribs[1].astype(jnp.bfloat16)
        rs0 = rs_step(0)
        rs0.start()
        ag1.wait_recv()
        ag2 = ag_hop(2)
        ag2.start()
        contribs[2] = contribution(2)

        rs0.wait_recv()
        sendbuf[1] = (contribs[2] + recvbuf[0].astype(jnp.float32)).astype(
            jnp.bfloat16)
        rs1 = rs_step(1)
        rs1.start()
        ag2.wait_recv()
        contribs[3] = contribution(3)

        rs1.wait_recv()
        sendbuf[2] = (contribs[3] + recvbuf[1].astype(jnp.float32)).astype(
            jnp.bfloat16)
        rs2 = rs_step(2)
        rs2.start()
        rs2.wait_recv()
        out_ref[...] = contribs[0] + recvbuf[N_DEV - 2].astype(jnp.float32)

        for d in (ag0, ag1, ag2, rs0, rs1, rs2):
            d.wait_send()

    x2 = x.reshape(ROWS, D)
    out = pl.pallas_call(
        body,
        out_shape=jax.ShapeDtypeStruct((ROWS, D), jnp.float32),
        in_specs=[pl.BlockSpec(memory_space=pltpu.VMEM)] * 5,
        out_specs=pl.BlockSpec(memory_space=pltpu.VMEM),
        scratch_shapes=[
            pltpu.VMEM((N_DEV, ROWS, D), jnp.bfloat16),
            pltpu.VMEM((N_DEV, ROWS, D), jnp.float32),
            pltpu.VMEM((N_DEV - 1, ROWS, D), jnp.bfloat16),
            pltpu.VMEM((N_DEV - 1, ROWS, D), jnp.bfloat16),
            pltpu.VMEM((D, 3 * D), jnp.bfloat16),
            pltpu.VMEM((D, D), jnp.bfloat16),
            pltpu.SemaphoreType.DMA((N_DEV - 1,)),
            pltpu.SemaphoreType.DMA((N_DEV - 1,)),
            pltpu.SemaphoreType.DMA((N_DEV - 1,)),
            pltpu.SemaphoreType.DMA((N_DEV - 1,)),
        ],
        compiler_params=pltpu.CompilerParams(collective_id=0),
    )(x2, Wq, Wo, Wk, Wv)
    return out.reshape(B_PER, SQ, D)


# device time: 24764 ns/iter; 1.9897x vs baseline; 1.3082x over previous
import jax
import jax.numpy as jnp
from jax import lax
from jax.experimental import pallas as pl
from jax.experimental.pallas import tpu as pltpu

N_DEV = 4
B_PER = 2
SQ = 128
D = 512
H_PER = 8
DH = 64
ROWS = B_PER * SQ


def kernel(x, Wq, Wo, Wk, Wv):
    def body(x_ref, wq_ref, wo_ref, wk_ref, wv_ref, out_ref,
             xa, xb, contribs, recva, senda, recvb, sendb,
             wqkv16, wo16,
             aga_s, aga_r, agb_s, agb_r, rsa_s, rsa_r, rsb_s, rsb_r):
        my = lax.axis_index("i")
        left = lax.rem(my + N_DEV - 1, N_DEV)
        right = lax.rem(my + 1, N_DEV)

        barrier = pltpu.get_barrier_semaphore()
        for nbr in (left, right):
            pl.semaphore_signal(
                barrier, inc=1, device_id=(nbr,),
                device_id_type=pl.DeviceIdType.MESH,
            )
        pl.semaphore_wait(barrier, 2)

        xa[0] = x_ref[0:SQ, :].astype(jnp.bfloat16)
        xb[0] = x_ref[SQ:ROWS, :].astype(jnp.bfloat16)

        def ag_a(h):
            return pltpu.make_async_remote_copy(
                src_ref=xa.at[h], dst_ref=xa.at[h + 1],
                send_sem=aga_s.at[h], recv_sem=aga_r.at[h],
                device_id=(right,), device_id_type=pl.DeviceIdType.MESH,
            )

        def ag_b(h):
            return pltpu.make_async_remote_copy(
                src_ref=xb.at[h], dst_ref=xb.at[h + 1],
                send_sem=agb_s.at[h], recv_sem=agb_r.at[h],
                device_id=(left,), device_id_type=pl.DeviceIdType.MESH,
            )

        def rs_a(s):
            return pltpu.make_async_remote_copy(
                src_ref=senda.at[s], dst_ref=recva.at[s],
                send_sem=rsa_s.at[s], recv_sem=rsa_r.at[s],
                device_id=(right,), device_id_type=pl.DeviceIdType.MESH,
            )

        def rs_b(s):
            return pltpu.make_async_remote_copy(
                src_ref=sendb.at[s], dst_ref=recvb.at[s],
                send_sem=rsb_s.at[s], recv_sem=rsb_r.at[s],
                device_id=(left,), device_id_type=pl.DeviceIdType.MESH,
            )

        wqkv16[:, 0 * D:1 * D] = wq_ref[...].astype(jnp.bfloat16)
        wqkv16[:, 1 * D:2 * D] = wk_ref[...].astype(jnp.bfloat16)
        wqkv16[:, 2 * D:3 * D] = wv_ref[...].astype(jnp.bfloat16)
        wo16[...] = wo_ref[...].astype(jnp.bfloat16)

        def contribution(r):
            xc = jnp.concatenate([xa[r], xb[r]], axis=0)
            qkv = jnp.dot(xc, wqkv16[...], preferred_element_type=jnp.float32)
            qkv16 = qkv.astype(jnp.bfloat16)

            def heads(i):
                parts = []
                for hh in range(H_PER):
                    csl = slice(i * D + hh * DH, i * D + (hh + 1) * DH)
                    parts.append(qkv16[:, csl].reshape(1, B_PER, SQ, DH))
                return jnp.concatenate(parts, axis=0).reshape(
                    H_PER * B_PER, SQ, DH)

            q4 = heads(0)
            k4 = heads(1)
            v4 = heads(2)
            s = lax.dot_general(
                q4, k4, (((2,), (2,)), ((0,), (0,))),
                preferred_element_type=jnp.float32,
            )
            p = jnp.exp(s * 0.125)
            lsum = jnp.sum(p, axis=-1, keepdims=True)
            o = lax.dot_general(
                p.astype(jnp.bfloat16), v4, (((2,), (1,)), ((0,), (0,))),
                preferred_element_type=jnp.float32,
            ) / lsum
            attn = o.reshape(H_PER, B_PER, SQ, DH).transpose(
                1, 2, 0, 3).reshape(ROWS, D)
            return jnp.dot(
                attn.astype(jnp.bfloat16), wo16[...],
                preferred_element_type=jnp.float32,
            )

        aga0, agb0 = ag_a(0), ag_b(0)
        aga0.start()
        agb0.start()
        contribs[0] = contribution(0)

        aga0.wait_recv()
        agb0.wait_recv()
        aga1, agb1 = ag_a(1), ag_b(1)
        aga1.start()
        agb1.start()
        contribs[1] = contribution(1)

        c1 = contribs[1]
        senda[0] = c1[0:SQ, :].astype(jnp.bfloat16)
        sendb[0] = c1[SQ:ROWS, :].astype(jnp.bfloat16)
        rsa0, rsb0 = rs_a(0), rs_b(0)
        rsa0.start()
        rsb0.start()
        aga1.wait_recv()
        agb1.wait_recv()
        aga2, agb2 = ag_a(2), ag_b(2)
        aga2.start()
        agb2.start()
        contribs[2] = contribution(2)

        rsa0.wait_recv()
        rsb0.wait_recv()
        c2 = contribs[2]
        senda[1] = (c2[0:SQ, :] + recva[0].astype(jnp.float32)).astype(
            jnp.bfloat16)
        sendb[1] = (c2[SQ:ROWS, :] + recvb[0].astype(jnp.float32)).astype(
            jnp.bfloat16)
        rsa1, rsb1 = rs_a(1), rs_b(1)
        rsa1.start()
        rsb1.start()
        aga2.wait_recv()
        agb2.wait_recv()
        contribs[3] = contribution(3)

        rsa1.wait_recv()
        rsb1.wait_recv()
        c3 = contribs[3]
        senda[2] = (c3[0:SQ, :] + recva[1].astype(jnp.float32)).astype(
            jnp.bfloat16)
        sendb[2] = (c3[SQ:ROWS, :] + recvb[1].astype(jnp.float32)).astype(
            jnp.bfloat16)
        rsa2, rsb2 = rs_a(2), rs_b(2)
        rsa2.start()
        rsb2.start()
        rsa2.wait_recv()
        rsb2.wait_recv()
        c0 = contribs[0]
        out_ref[0:SQ, :] = c0[0:SQ, :] + recva[2].astype(jnp.float32)
        out_ref[SQ:ROWS, :] = c0[SQ:ROWS, :] + recvb[2].astype(jnp.float32)

        for d in (aga0, aga1, aga2, agb0, agb1, agb2,
                  rsa0, rsa1, rsa2, rsb0, rsb1, rsb2):
            d.wait_send()

    x2 = x.reshape(ROWS, D)
    out = pl.pallas_call(
        body,
        out_shape=jax.ShapeDtypeStruct((ROWS, D), jnp.float32),
        in_specs=[pl.BlockSpec(memory_space=pltpu.VMEM)] * 5,
        out_specs=pl.BlockSpec(memory_space=pltpu.VMEM),
        scratch_shapes=[
            pltpu.VMEM((N_DEV, SQ, D), jnp.bfloat16),
            pltpu.VMEM((N_DEV, SQ, D), jnp.bfloat16),
            pltpu.VMEM((N_DEV, ROWS, D), jnp.float32),
            pltpu.VMEM((N_DEV - 1, SQ, D), jnp.bfloat16),
            pltpu.VMEM((N_DEV - 1, SQ, D), jnp.bfloat16),
            pltpu.VMEM((N_DEV - 1, SQ, D), jnp.bfloat16),
            pltpu.VMEM((N_DEV - 1, SQ, D), jnp.bfloat16),
            pltpu.VMEM((D, 3 * D), jnp.bfloat16),
            pltpu.VMEM((D, D), jnp.bfloat16),
            pltpu.SemaphoreType.DMA((N_DEV - 1,)),
            pltpu.SemaphoreType.DMA((N_DEV - 1,)),
            pltpu.SemaphoreType.DMA((N_DEV - 1,)),
            pltpu.SemaphoreType.DMA((N_DEV - 1,)),
            pltpu.SemaphoreType.DMA((N_DEV - 1,)),
            pltpu.SemaphoreType.DMA((N_DEV - 1,)),
            pltpu.SemaphoreType.DMA((N_DEV - 1,)),
            pltpu.SemaphoreType.DMA((N_DEV - 1,)),
        ],
        compiler_params=pltpu.CompilerParams(collective_id=0),
    )(x2, Wq, Wo, Wk, Wv)
    return out.reshape(B_PER, SQ, D)
